# Initial kernel scaffold; baseline (speedup 1.0000x reference)
import jax
import jax.numpy as jnp
from jax.experimental import pallas as pl


def kernel(x, table):
    raise NotImplementedError("write your pallas kernel here")


# speedup vs baseline: 1.1235x; 1.1235x over previous
import jax
import jax.numpy as jnp
from jax.experimental import pallas as pl

MAX_LEN = 8192


def _add_kernel(x_ref, t_ref, o_ref):
    o_ref[...] = x_ref[...] + t_ref[...]


def kernel(x, table):
    bsz, seq_len, d = x.shape
    if seq_len > MAX_LEN:
        x = x[:, -MAX_LEN:, :]
        seq_len = MAX_LEN
    bs = 256
    grid = (seq_len // bs, bsz)
    return pl.pallas_call(
        _add_kernel,
        grid=grid,
        in_specs=[
            pl.BlockSpec((1, bs, d), lambda j, b: (b, j, 0)),
            pl.BlockSpec((bs, d), lambda j, b: (j, 0)),
        ],
        out_specs=pl.BlockSpec((1, bs, d), lambda j, b: (b, j, 0)),
        out_shape=jax.ShapeDtypeStruct(x.shape, x.dtype),
    )(x, table)


# speedup vs baseline: 1.7221x; 1.5328x over previous
import jax
import jax.numpy as jnp
from jax.experimental import pallas as pl

MAX_LEN = 8192


def _add_kernel(x_ref, t_ref, o_ref):
    o_ref[...] = x_ref[...] + t_ref[...]


def kernel(x, table):
    bsz, seq_len, d = x.shape
    if seq_len > MAX_LEN:
        x = x[:, -MAX_LEN:, :]
        seq_len = MAX_LEN
    bs = 256
    grid = (seq_len // bs,)
    return pl.pallas_call(
        _add_kernel,
        grid=grid,
        in_specs=[
            pl.BlockSpec((bsz, bs, d), lambda j: (0, j, 0)),
            pl.BlockSpec((bs, d), lambda j: (j, 0)),
        ],
        out_specs=pl.BlockSpec((bsz, bs, d), lambda j: (0, j, 0)),
        out_shape=jax.ShapeDtypeStruct(x.shape, x.dtype),
    )(x, table)
